# Initial kernel scaffold; baseline (speedup 1.0000x reference)
import jax
import jax.numpy as jnp
from jax.experimental import pallas as pl


def kernel(user_emb, item_emb, edge_index, edge_values):
    raise NotImplementedError("write your pallas kernel here")


# speedup vs baseline: 5.0464x; 5.0464x over previous
import jax
import jax.numpy as jnp
from jax import lax
from jax.experimental import pallas as pl
from jax.experimental.pallas import tpu as pltpu
from jax.experimental.pallas import tpu_sc as plsc

N_USERS = 50000
N_ITEMS = 50000
N = N_USERS + N_ITEMS
NP = 100352
H = 16
E = 1600000
NC = 2
NS = 16
CHUNK = 128
B = 8
GROUP = B * CHUNK
E_PER_TILE = -(-E // (NS * GROUP)) * GROUP
E_PAD = E_PER_TILE * NS
N_GROUPS = E_PER_TILE // GROUP
ROWS_PER_TILE = NP // NS
RCHUNK = 128
N_RCHUNKS = ROWS_PER_TILE // RCHUNK
N_LAYERS = 3


def _propagate_body(t0_r, dst_r, src_r, val_r,
                    out_r, tab_r,
                    dstA, srcA, valA,
                    dstB, srcB, valB,
                    rows,
                    acc,
                    sem_in, sem_g, sem_s):
    c = lax.axis_index("c")
    s = lax.axis_index("s")
    coff = c * NP
    row0 = s * ROWS_PER_TILE
    ii = lax.broadcasted_iota(jnp.int32, (16,), 0)
    zero16 = jnp.zeros((16,), jnp.float32)
    zb = rows.at[0]
    ebase = s * E_PER_TILE

    def fill_zb():
        def _z(i, _):
            zb[i] = zero16
            return 0
        lax.fori_loop(0, RCHUNK, _z, 0)

    def _init(k, _):
        r = row0 + k * RCHUNK
        pltpu.sync_copy(t0_r.at[pl.ds(coff + r, RCHUNK)], zb)
        pltpu.sync_copy(zb, tab_r.at[pl.ds(coff + r, RCHUNK)])
        return 0
    lax.fori_loop(0, N_RCHUNKS, _init, 0)

    def layer(l, _):
        fill_zb()

        def _za(k, _):
            pltpu.sync_copy(zb, acc.at[pl.ds(row0 + k * RCHUNK, RCHUNK)])
            return 0
        lax.fori_loop(0, N_RCHUNKS, _za, 0)
        plsc.subcore_barrier()

        sbase = (l * NC + c) * E_PAD + ebase

        def issue_in(g, db, sb, vb):
            o = g * GROUP
            for b in range(B):
                off = o + b * CHUNK
                pltpu.async_copy(
                    dst_r.at[pl.ds(ebase + off, CHUNK)], db.at[b], sem_in)
                pltpu.async_copy(
                    src_r.at[pl.ds(sbase + off, CHUNK)], sb.at[b], sem_in)
                pltpu.async_copy(
                    val_r.at[pl.ds(ebase + off, CHUNK)], vb.at[b], sem_in)

        def drain_in(db, sb, vb):
            for b in range(B):
                pltpu.make_async_copy(
                    dst_r.at[pl.ds(ebase, CHUNK)], db.at[b], sem_in).wait()
                pltpu.make_async_copy(
                    src_r.at[pl.ds(sbase, CHUNK)], sb.at[b], sem_in).wait()
                pltpu.make_async_copy(
                    val_r.at[pl.ds(ebase, CHUNK)], vb.at[b], sem_in).wait()

        def gathers(sb):
            return [pltpu.async_copy(tab_r.at[sb.at[b]], rows.at[b], sem_g)
                    for b in range(B)]

        def mult_scatter(db, vb):
            d_s = []
            for b in range(B):
                rslot = rows.at[b]
                vslot = vb.at[b]
                for g in range(CHUNK // 16):
                    valg = vslot[pl.ds(g * 16, 16)]
                    ridx = ii + g * 16
                    for col in range(H):
                        cidx = jnp.full((16,), col, jnp.int32)
                        v = plsc.load_gather(rslot, [ridx, cidx])
                        plsc.store_scatter(rslot, [ridx, cidx], v * valg)
                d_s.append(pltpu.async_copy(
                    rslot, acc.at[db.at[b]], sem_s, add=True))
            for d in d_s:
                d.wait()

        def half(g_cur, g_next, db, sb, vb, db2, sb2, vb2, prefetch):
            drain_in(db, sb, vb)
            d_g = gathers(sb)
            if prefetch is None:
                issue_in(g_next, db2, sb2, vb2)
            else:
                @pl.when(prefetch)
                def _():
                    issue_in(g_next, db2, sb2, vb2)
            for d in d_g:
                d.wait()
            mult_scatter(db, vb)

        issue_in(0, dstA, srcA, valA)

        def pair(pi, _):
            g0 = 2 * pi
            half(g0, g0 + 1, dstA, srcA, valA, dstB, srcB, valB, None)
            half(g0 + 1, g0 + 2, dstB, srcB, valB, dstA, srcA, valA,
                 pi + 1 < N_GROUPS // 2)
            return 0

        lax.fori_loop(0, N_GROUPS // 2, pair, 0)
        plsc.subcore_barrier()

        tb = (l + 1) * (NC * NP) + coff + row0

        def _wb(k, _):
            pltpu.sync_copy(acc.at[pl.ds(row0 + k * RCHUNK, RCHUNK)],
                            tab_r.at[pl.ds(tb + k * RCHUNK, RCHUNK)])
            return 0
        lax.fori_loop(0, N_RCHUNKS, _wb, 0)
        plsc.subcore_barrier()
        return 0

    lax.fori_loop(0, N_LAYERS, layer, 0)

    b0, b1, b2, b3 = rows.at[0], rows.at[1], rows.at[2], rows.at[3]

    def _mean(k, _):
        hb = coff + row0 + k * RCHUNK
        S = NC * NP
        pltpu.sync_copy(tab_r.at[pl.ds(hb, RCHUNK)], b0)
        pltpu.sync_copy(tab_r.at[pl.ds(S + hb, RCHUNK)], b1)
        pltpu.sync_copy(tab_r.at[pl.ds(2 * S + hb, RCHUNK)], b2)
        pltpu.sync_copy(tab_r.at[pl.ds(3 * S + hb, RCHUNK)], b3)

        def _m(j, _):
            b0[j] = (b1[j] + b2[j] + b3[j] + b0[j]) * 0.25
            return 0
        lax.fori_loop(0, RCHUNK, _m, 0)
        pltpu.sync_copy(b0, out_r.at[pl.ds(hb, RCHUNK)])
        return 0
    lax.fori_loop(0, N_RCHUNKS, _mean, 0)


@jax.jit
def _propagate(t0, dst, src3, val):
    mesh = plsc.VectorSubcoreMesh(core_axis_name="c", subcore_axis_name="s",
                                  num_cores=NC, num_subcores=NS)
    f32 = jnp.float32
    run = pl.kernel(
        _propagate_body,
        out_type=(
            jax.ShapeDtypeStruct((NC * NP, H), f32),
            jax.ShapeDtypeStruct(((N_LAYERS + 1) * NC * NP, H), f32),
        ),
        mesh=mesh,
        compiler_params=pltpu.CompilerParams(use_tc_tiling_on_sc=False,
                                             needs_layout_passes=False),
        scratch_types=(
            pltpu.VMEM((B, CHUNK), jnp.int32),
            pltpu.VMEM((B, CHUNK), jnp.int32),
            pltpu.VMEM((B, CHUNK), f32),
            pltpu.VMEM((B, CHUNK), jnp.int32),
            pltpu.VMEM((B, CHUNK), jnp.int32),
            pltpu.VMEM((B, CHUNK), f32),
            pltpu.VMEM((B, CHUNK, H), f32),
            pltpu.VMEM_SHARED((NP, H), f32),
            pltpu.SemaphoreType.DMA,
            pltpu.SemaphoreType.DMA,
            pltpu.SemaphoreType.DMA,
        ),
    )
    out_sum, _tab = run(t0, dst, src3, val)
    return out_sum


def kernel(user_emb, item_emb, edge_index, edge_values):
    all0 = jnp.concatenate([user_emb, item_emb], axis=0)
    zrows = jnp.zeros((NP - N, H), jnp.float32)
    t0 = jnp.concatenate([all0[:, :H], zrows, all0[:, H:], zrows], axis=0)
    dst = edge_index[0].astype(jnp.int32)
    src = edge_index[1].astype(jnp.int32)
    val = edge_values.astype(jnp.float32)
    pad = E_PAD - E
    dst = jnp.pad(dst, (0, pad))
    src = jnp.pad(src, (0, pad))
    val = jnp.pad(val, (0, pad))
    offs = jnp.arange(N_LAYERS * NC, dtype=jnp.int32) * NP
    src3 = (src[None, :] + offs[:, None]).reshape(-1)
    mean = _propagate(t0, dst, src3, val)
    all_emb = jnp.concatenate([mean[:N], mean[NP:NP + N]], axis=1)
    return (all_emb[:N_USERS], all_emb[N_USERS:])


# speedup vs baseline: 5.7541x; 1.1403x over previous
import jax
import jax.numpy as jnp
from jax import lax
from jax.experimental import pallas as pl
from jax.experimental.pallas import tpu as pltpu
from jax.experimental.pallas import tpu_sc as plsc

N_USERS = 50000
N_ITEMS = 50000
N = N_USERS + N_ITEMS
NP = 100352
H = 16
E = 1600000
NC = 2
NS = 16
CHUNK = 128
B = 8
GROUP = B * CHUNK
E_PER_TILE = -(-E // (NS * GROUP)) * GROUP
E_PAD = E_PER_TILE * NS
N_GROUPS = E_PER_TILE // GROUP
ROWS_PER_TILE = NP // NS
RCHUNK = 128
N_RCHUNKS = ROWS_PER_TILE // RCHUNK
N_LAYERS = 3


def _propagate_body(t0_r, dst_r, src_r, val_r,
                    out_r, tab_r,
                    dstA, srcA, valA,
                    dstB, srcB, valB,
                    rows,
                    acc,
                    sem_in, sem_g, sem_s):
    c = lax.axis_index("c")
    s = lax.axis_index("s")
    coff = c * NP
    row0 = s * ROWS_PER_TILE
    ii = lax.broadcasted_iota(jnp.int32, (16,), 0)
    zero16 = jnp.zeros((16,), jnp.float32)
    zb = rows.at[0]
    ebase = s * E_PER_TILE

    def fill_zb():
        def _z(i, _):
            zb[i] = zero16
            return 0
        lax.fori_loop(0, RCHUNK, _z, 0)

    def _init(k, _):
        r = row0 + k * RCHUNK
        pltpu.sync_copy(t0_r.at[pl.ds(coff + r, RCHUNK)], zb)
        pltpu.sync_copy(zb, tab_r.at[pl.ds(coff + r, RCHUNK)])
        return 0
    lax.fori_loop(0, N_RCHUNKS, _init, 0)

    def layer(l, _):
        fill_zb()

        def _za(k, _):
            pltpu.sync_copy(zb, acc.at[pl.ds(row0 + k * RCHUNK, RCHUNK)])
            return 0
        lax.fori_loop(0, N_RCHUNKS, _za, 0)
        plsc.subcore_barrier()

        sbase = c * E_PAD + ebase
        tin = tab_r.at[pl.ds(l * (NC * NP), NC * NP)]

        def issue_in(g, db, sb, vb):
            o = g * GROUP
            for b in range(B):
                off = o + b * CHUNK
                pltpu.async_copy(
                    dst_r.at[pl.ds(ebase + off, CHUNK)], db.at[b], sem_in)
                pltpu.async_copy(
                    src_r.at[pl.ds(sbase + off, CHUNK)], sb.at[b], sem_in)
                pltpu.async_copy(
                    val_r.at[pl.ds(ebase + off, CHUNK)], vb.at[b], sem_in)

        def drain_in(db, sb, vb):
            for b in range(B):
                pltpu.make_async_copy(
                    dst_r.at[pl.ds(ebase, CHUNK)], db.at[b], sem_in).wait()
                pltpu.make_async_copy(
                    src_r.at[pl.ds(sbase, CHUNK)], sb.at[b], sem_in).wait()
                pltpu.make_async_copy(
                    val_r.at[pl.ds(ebase, CHUNK)], vb.at[b], sem_in).wait()

        def gathers(sb):
            return [pltpu.async_copy(tin.at[sb.at[b]], rows.at[b], sem_g)
                    for b in range(B)]

        def mult_scatter(db, vb):
            d_s = []
            for b in range(B):
                rslot = rows.at[b]
                vslot = vb.at[b]
                for g in range(CHUNK // 16):
                    valg = vslot[pl.ds(g * 16, 16)]
                    ridx = ii + g * 16
                    for col in range(H):
                        cidx = jnp.full((16,), col, jnp.int32)
                        v = plsc.load_gather(rslot, [ridx, cidx])
                        plsc.store_scatter(rslot, [ridx, cidx], v * valg)
                d_s.append(pltpu.async_copy(
                    rslot, acc.at[db.at[b]], sem_s, add=True))
            for d in d_s:
                d.wait()

        def half(g_cur, g_next, db, sb, vb, db2, sb2, vb2, prefetch):
            drain_in(db, sb, vb)
            d_g = gathers(sb)
            if prefetch is None:
                issue_in(g_next, db2, sb2, vb2)
            else:
                @pl.when(prefetch)
                def _():
                    issue_in(g_next, db2, sb2, vb2)
            for d in d_g:
                d.wait()
            mult_scatter(db, vb)

        issue_in(0, dstA, srcA, valA)

        def pair(pi, _):
            g0 = 2 * pi
            half(g0, g0 + 1, dstA, srcA, valA, dstB, srcB, valB, None)
            half(g0 + 1, g0 + 2, dstB, srcB, valB, dstA, srcA, valA,
                 pi + 1 < N_GROUPS // 2)
            return 0

        lax.fori_loop(0, N_GROUPS // 2, pair, 0)
        plsc.subcore_barrier()

        tb = (l + 1) * (NC * NP) + coff + row0

        def _wb(k, _):
            pltpu.sync_copy(acc.at[pl.ds(row0 + k * RCHUNK, RCHUNK)],
                            tab_r.at[pl.ds(tb + k * RCHUNK, RCHUNK)])
            return 0
        lax.fori_loop(0, N_RCHUNKS, _wb, 0)
        plsc.subcore_barrier()
        return 0

    lax.fori_loop(0, N_LAYERS, layer, 0)

    b0, b1, b2, b3 = rows.at[0], rows.at[1], rows.at[2], rows.at[3]

    def _mean(k, _):
        hb = coff + row0 + k * RCHUNK
        S = NC * NP
        pltpu.sync_copy(tab_r.at[pl.ds(hb, RCHUNK)], b0)
        pltpu.sync_copy(tab_r.at[pl.ds(S + hb, RCHUNK)], b1)
        pltpu.sync_copy(tab_r.at[pl.ds(2 * S + hb, RCHUNK)], b2)
        pltpu.sync_copy(tab_r.at[pl.ds(3 * S + hb, RCHUNK)], b3)

        def _m(j, _):
            b0[j] = (b1[j] + b2[j] + b3[j] + b0[j]) * 0.25
            return 0
        lax.fori_loop(0, RCHUNK, _m, 0)
        pltpu.sync_copy(b0, out_r.at[pl.ds(hb, RCHUNK)])
        return 0
    lax.fori_loop(0, N_RCHUNKS, _mean, 0)


@jax.jit
def _propagate(t0, dst, src2, val):
    mesh = plsc.VectorSubcoreMesh(core_axis_name="c", subcore_axis_name="s",
                                  num_cores=NC, num_subcores=NS)
    f32 = jnp.float32
    run = pl.kernel(
        _propagate_body,
        out_type=(
            jax.ShapeDtypeStruct((NC * NP, H), f32),
            jax.ShapeDtypeStruct(((N_LAYERS + 1) * NC * NP, H), f32),
        ),
        mesh=mesh,
        compiler_params=pltpu.CompilerParams(use_tc_tiling_on_sc=False,
                                             needs_layout_passes=False),
        scratch_types=(
            pltpu.VMEM((B, CHUNK), jnp.int32),
            pltpu.VMEM((B, CHUNK), jnp.int32),
            pltpu.VMEM((B, CHUNK), f32),
            pltpu.VMEM((B, CHUNK), jnp.int32),
            pltpu.VMEM((B, CHUNK), jnp.int32),
            pltpu.VMEM((B, CHUNK), f32),
            pltpu.VMEM((B, CHUNK, H), f32),
            pltpu.VMEM_SHARED((NP, H), f32),
            pltpu.SemaphoreType.DMA,
            pltpu.SemaphoreType.DMA,
            pltpu.SemaphoreType.DMA,
        ),
    )
    out_sum, _tab = run(t0, dst, src2, val)
    return out_sum


def kernel(user_emb, item_emb, edge_index, edge_values):
    all0 = jnp.concatenate([user_emb, item_emb], axis=0)
    zrows = jnp.zeros((NP - N, H), jnp.float32)
    t0 = jnp.concatenate([all0[:, :H], zrows, all0[:, H:], zrows], axis=0)
    dst = edge_index[0].astype(jnp.int32)
    src = edge_index[1].astype(jnp.int32)
    val = edge_values.astype(jnp.float32)
    pad = E_PAD - E
    dst = jnp.pad(dst, (0, pad))
    src = jnp.pad(src, (0, pad))
    val = jnp.pad(val, (0, pad))
    src2 = jnp.concatenate([src, src + NP])
    mean = _propagate(t0, dst, src2, val)
    all_emb = jnp.concatenate([mean[:N], mean[NP:NP + N]], axis=1)
    return (all_emb[:N_USERS], all_emb[N_USERS:])


# speedup vs baseline: 13.9798x; 2.4295x over previous
import jax
import jax.numpy as jnp
from jax import lax
from jax.experimental import pallas as pl
from jax.experimental.pallas import tpu as pltpu
from jax.experimental.pallas import tpu_sc as plsc

N_USERS = 50000
N_ITEMS = 50000
N = N_USERS + N_ITEMS
NP = 100352
H = 16
E = 1600000
NC = 2
NS = 16
CHUNK = 128
B = 8
GROUP = B * CHUNK
E_PER_TILE = -(-E // (NS * GROUP)) * GROUP
E_PAD = E_PER_TILE * NS
N_GROUPS = E_PER_TILE // GROUP
ROWS_PER_TILE = NP // NS
RCHUNK = 128
N_RCHUNKS = ROWS_PER_TILE // RCHUNK
N_LAYERS = 3


def _propagate_body(t0_r, dst_r, src_r, val_r,
                    out_r, tab_r,
                    dstA, srcA, valA,
                    dstB, srcB, valB,
                    rows,
                    acc,
                    sem_in, sem_g, sem_s):
    c = lax.axis_index("c")
    s = lax.axis_index("s")
    coff = c * NP
    row0 = s * ROWS_PER_TILE
    ii = lax.broadcasted_iota(jnp.int32, (16,), 0)
    zero16 = jnp.zeros((16,), jnp.float32)
    zb = rows.at[0]
    ebase = s * E_PER_TILE

    def fill_zb():
        def _z(i, _):
            zb[i] = zero16
            return 0
        lax.fori_loop(0, RCHUNK, _z, 0)

    def _init(k, _):
        r = row0 + k * RCHUNK
        pltpu.sync_copy(t0_r.at[pl.ds(coff + r, RCHUNK)], zb)
        pltpu.sync_copy(zb, tab_r.at[pl.ds(coff + r, RCHUNK)])
        return 0
    lax.fori_loop(0, N_RCHUNKS, _init, 0)

    def layer(l, _):
        fill_zb()

        def _za(k, _):
            pltpu.sync_copy(zb, acc.at[pl.ds(row0 + k * RCHUNK, RCHUNK)])
            return 0
        lax.fori_loop(0, N_RCHUNKS, _za, 0)
        plsc.subcore_barrier()

        sbase = c * E_PAD + ebase
        tin = tab_r.at[pl.ds(l * (NC * NP), NC * NP)]

        def issue_in(g, db, sb, vb):
            o = g * GROUP
            for b in range(B):
                off = o + b * CHUNK
                pltpu.async_copy(
                    dst_r.at[pl.ds(ebase + off, CHUNK)], db.at[b], sem_in)
                pltpu.async_copy(
                    src_r.at[pl.ds(sbase + off, CHUNK)], sb.at[b], sem_in)
                pltpu.async_copy(
                    val_r.at[pl.ds(ebase + off, CHUNK)], vb.at[b], sem_in)

        def drain_in(db, sb, vb):
            for b in range(B):
                pltpu.make_async_copy(
                    dst_r.at[pl.ds(ebase, CHUNK)], db.at[b], sem_in).wait()
                pltpu.make_async_copy(
                    src_r.at[pl.ds(sbase, CHUNK)], sb.at[b], sem_in).wait()
                pltpu.make_async_copy(
                    val_r.at[pl.ds(ebase, CHUNK)], vb.at[b], sem_in).wait()

        def gathers(sb):
            return [pltpu.async_copy(tin.at[sb.at[b]], rows.at[b], sem_g)
                    for b in range(B)]

        def mult_scatter(db, vb):
            d_s = []
            for b in range(B):
                rslot = rows.at[b]
                vslot = vb.at[b]
                for g in range(CHUNK // 16):
                    valg = vslot[pl.ds(g * 16, 16)]
                    for j in range(16):
                        e = g * 16 + j
                        vbc = jnp.full((16,), valg[j])
                        rslot[e] = rslot[e] * vbc
                d_s.append(pltpu.async_copy(
                    rslot, acc.at[db.at[b]], sem_s, add=True))
            for d in d_s:
                d.wait()

        def half(g_cur, g_next, db, sb, vb, db2, sb2, vb2, prefetch):
            drain_in(db, sb, vb)
            d_g = gathers(sb)
            if prefetch is None:
                issue_in(g_next, db2, sb2, vb2)
            else:
                @pl.when(prefetch)
                def _():
                    issue_in(g_next, db2, sb2, vb2)
            for d in d_g:
                d.wait()
            mult_scatter(db, vb)

        issue_in(0, dstA, srcA, valA)

        def pair(pi, _):
            g0 = 2 * pi
            half(g0, g0 + 1, dstA, srcA, valA, dstB, srcB, valB, None)
            half(g0 + 1, g0 + 2, dstB, srcB, valB, dstA, srcA, valA,
                 pi + 1 < N_GROUPS // 2)
            return 0

        lax.fori_loop(0, N_GROUPS // 2, pair, 0)
        plsc.subcore_barrier()

        tb = (l + 1) * (NC * NP) + coff + row0

        def _wb(k, _):
            pltpu.sync_copy(acc.at[pl.ds(row0 + k * RCHUNK, RCHUNK)],
                            tab_r.at[pl.ds(tb + k * RCHUNK, RCHUNK)])
            return 0
        lax.fori_loop(0, N_RCHUNKS, _wb, 0)
        plsc.subcore_barrier()
        return 0

    lax.fori_loop(0, N_LAYERS, layer, 0)

    b0, b1, b2, b3 = rows.at[0], rows.at[1], rows.at[2], rows.at[3]

    def _mean(k, _):
        hb = coff + row0 + k * RCHUNK
        S = NC * NP
        pltpu.sync_copy(tab_r.at[pl.ds(hb, RCHUNK)], b0)
        pltpu.sync_copy(tab_r.at[pl.ds(S + hb, RCHUNK)], b1)
        pltpu.sync_copy(tab_r.at[pl.ds(2 * S + hb, RCHUNK)], b2)
        pltpu.sync_copy(tab_r.at[pl.ds(3 * S + hb, RCHUNK)], b3)

        def _m(j, _):
            b0[j] = (b1[j] + b2[j] + b3[j] + b0[j]) * 0.25
            return 0
        lax.fori_loop(0, RCHUNK, _m, 0)
        pltpu.sync_copy(b0, out_r.at[pl.ds(hb, RCHUNK)])
        return 0
    lax.fori_loop(0, N_RCHUNKS, _mean, 0)


@jax.jit
def _propagate(t0, dst, src2, val):
    mesh = plsc.VectorSubcoreMesh(core_axis_name="c", subcore_axis_name="s",
                                  num_cores=NC, num_subcores=NS)
    f32 = jnp.float32
    run = pl.kernel(
        _propagate_body,
        out_type=(
            jax.ShapeDtypeStruct((NC * NP, H), f32),
            jax.ShapeDtypeStruct(((N_LAYERS + 1) * NC * NP, H), f32),
        ),
        mesh=mesh,
        compiler_params=pltpu.CompilerParams(use_tc_tiling_on_sc=False,
                                             needs_layout_passes=False),
        scratch_types=(
            pltpu.VMEM((B, CHUNK), jnp.int32),
            pltpu.VMEM((B, CHUNK), jnp.int32),
            pltpu.VMEM((B, CHUNK), f32),
            pltpu.VMEM((B, CHUNK), jnp.int32),
            pltpu.VMEM((B, CHUNK), jnp.int32),
            pltpu.VMEM((B, CHUNK), f32),
            pltpu.VMEM((B, CHUNK, H), f32),
            pltpu.VMEM_SHARED((NP, H), f32),
            pltpu.SemaphoreType.DMA,
            pltpu.SemaphoreType.DMA,
            pltpu.SemaphoreType.DMA,
        ),
    )
    out_sum, _tab = run(t0, dst, src2, val)
    return out_sum


def kernel(user_emb, item_emb, edge_index, edge_values):
    all0 = jnp.concatenate([user_emb, item_emb], axis=0)
    zrows = jnp.zeros((NP - N, H), jnp.float32)
    t0 = jnp.concatenate([all0[:, :H], zrows, all0[:, H:], zrows], axis=0)
    dst = edge_index[0].astype(jnp.int32)
    src = edge_index[1].astype(jnp.int32)
    val = edge_values.astype(jnp.float32)
    pad = E_PAD - E
    dst = jnp.pad(dst, (0, pad))
    src = jnp.pad(src, (0, pad))
    val = jnp.pad(val, (0, pad))
    src2 = jnp.concatenate([src, src + NP])
    mean = _propagate(t0, dst, src2, val)
    all_emb = jnp.concatenate([mean[:N], mean[NP:NP + N]], axis=1)
    return (all_emb[:N_USERS], all_emb[N_USERS:])


# speedup vs baseline: 16.8774x; 1.2073x over previous
import jax
import jax.numpy as jnp
from jax import lax
from jax.experimental import pallas as pl
from jax.experimental.pallas import tpu as pltpu
from jax.experimental.pallas import tpu_sc as plsc

N_USERS = 50000
N_ITEMS = 50000
N = N_USERS + N_ITEMS
NP = 100352
H = 16
D = 32
E = 1600000
NC = 2
NS = 16
CHUNK = 128
B = 8
GROUP = B * CHUNK
E_PER_TILE = -(-E // (NS * GROUP)) * GROUP
E_PAD = E_PER_TILE * NS
N_GROUPS = E_PER_TILE // GROUP
ROWS_PER_TILE = NP // NS
S = NC * NP
N_STAGES = 4
INITC = 896
MEANC = 224


def _propagate_body(t0_r, dst_r, src_r, val_r,
                    out_r, tab_r,
                    dstA, srcA, valA,
                    dstB, srcB, valB,
                    rows,
                    zbuf,
                    acc,
                    sem_in, sem_g, sem_s):
    c = lax.axis_index("c")
    s = lax.axis_index("s")
    coff = c * NP
    row0 = s * ROWS_PER_TILE
    zero16 = jnp.zeros((16,), jnp.float32)
    ebase = s * E_PER_TILE
    cs = c * H

    def _z(i, _):
        zbuf[i] = zero16
        return 0
    lax.fori_loop(0, CHUNK, _z, 0)

    for k in range(ROWS_PER_TILE // INITC):
        r = row0 + k * INITC
        stage = rows.at[pl.ds(0, INITC)]
        pltpu.sync_copy(t0_r.at[pl.ds(r, INITC), pl.ds(cs, H)], stage)
        pltpu.sync_copy(stage, tab_r.at[pl.ds(coff + r, INITC)])

    def fire_drain(mk_src, mk_dst, n, sem):
        def _f(k, _):
            pltpu.async_copy(mk_src(k), mk_dst(k), sem)
            return 0
        lax.fori_loop(0, n, _f, 0)

        def _w(k, _):
            pltpu.make_async_copy(mk_src(k), mk_dst(k), sem).wait()
            return 0
        lax.fori_loop(0, n, _w, 0)

    n_zc = ROWS_PER_TILE // CHUNK

    def zero_acc():
        fire_drain(lambda k: zbuf,
                   lambda k: acc.at[pl.ds(row0 + k * CHUNK, CHUNK)],
                   n_zc, sem_s)

    def writeback(l):
        tb = (l + 1) * S + coff + row0
        fire_drain(lambda k: acc.at[pl.ds(row0 + k * CHUNK, CHUNK)],
                   lambda k: tab_r.at[pl.ds(tb + k * CHUNK, CHUNK)],
                   n_zc, sem_s)

    def layer(l, _):
        zero_acc()
        plsc.subcore_barrier()

        sbase = c * E_PAD + ebase
        tin = tab_r.at[pl.ds(l * S, S)]

        def issue_in(g, db, sb, vb):
            o = g * GROUP
            for b in range(B):
                off = o + b * CHUNK
                pltpu.async_copy(
                    dst_r.at[pl.ds(ebase + off, CHUNK)], db.at[b], sem_in)
                pltpu.async_copy(
                    src_r.at[pl.ds(sbase + off, CHUNK)], sb.at[b], sem_in)
                pltpu.async_copy(
                    val_r.at[pl.ds(ebase + off, CHUNK)], vb.at[b], sem_in)

        def drain_in(db, sb, vb):
            for b in range(B):
                pltpu.make_async_copy(
                    dst_r.at[pl.ds(ebase, CHUNK)], db.at[b], sem_in).wait()
                pltpu.make_async_copy(
                    src_r.at[pl.ds(sbase, CHUNK)], sb.at[b], sem_in).wait()
                pltpu.make_async_copy(
                    val_r.at[pl.ds(ebase, CHUNK)], vb.at[b], sem_in).wait()

        def gathers(sb):
            return [pltpu.async_copy(tin.at[sb.at[b]],
                                     rows.at[pl.ds(b * CHUNK, CHUNK)], sem_g)
                    for b in range(B)]

        def mult_scatter(db, vb):
            d_s = []
            for b in range(B):
                vslot = vb.at[b]
                for g in range(CHUNK // 16):
                    valg = vslot[pl.ds(g * 16, 16)]
                    for j in range(16):
                        e = b * CHUNK + g * 16 + j
                        vbc = jnp.full((16,), valg[j])
                        rows[e] = rows[e] * vbc
                d_s.append(pltpu.async_copy(
                    rows.at[pl.ds(b * CHUNK, CHUNK)],
                    acc.at[db.at[b]], sem_s, add=True))
            for d in d_s:
                d.wait()

        def half(g_next, db, sb, vb, db2, sb2, vb2, prefetch):
            drain_in(db, sb, vb)
            d_g = gathers(sb)
            if prefetch is None:
                issue_in(g_next, db2, sb2, vb2)
            else:
                @pl.when(prefetch)
                def _():
                    issue_in(g_next, db2, sb2, vb2)
            for d in d_g:
                d.wait()
            mult_scatter(db, vb)

        issue_in(0, dstA, srcA, valA)

        def pair(pi, _):
            g0 = 2 * pi
            half(g0 + 1, dstA, srcA, valA, dstB, srcB, valB, None)
            half(g0 + 2, dstB, srcB, valB, dstA, srcA, valA,
                 pi + 1 < N_GROUPS // 2)
            return 0

        lax.fori_loop(0, N_GROUPS // 2, pair, 0)
        plsc.subcore_barrier()

        writeback(l)
        plsc.subcore_barrier()
        return 0

    lax.fori_loop(0, N_LAYERS, layer, 0)

    def _mean(k, _):
        r = row0 + k * MEANC
        hb = coff + r
        d = []
        for l in range(N_STAGES):
            d.append(pltpu.async_copy(
                tab_r.at[pl.ds(l * S + hb, MEANC)],
                rows.at[pl.ds(l * MEANC, MEANC)], sem_in))
        for x in d:
            x.wait()

        def _m(j, _):
            acc_v = (rows[j] + rows[MEANC + j] +
                     rows[2 * MEANC + j] + rows[3 * MEANC + j]) * 0.25
            rows[j] = acc_v
            return 0
        lax.fori_loop(0, MEANC, _m, 0)
        pltpu.sync_copy(rows.at[pl.ds(0, MEANC)],
                        out_r.at[pl.ds(r, MEANC), pl.ds(cs, H)])
        return 0
    lax.fori_loop(0, ROWS_PER_TILE // MEANC, _mean, 0)


N_LAYERS = 3


@jax.jit
def _propagate(t0, dst, src2, val):
    mesh = plsc.VectorSubcoreMesh(core_axis_name="c", subcore_axis_name="s",
                                  num_cores=NC, num_subcores=NS)
    f32 = jnp.float32
    run = pl.kernel(
        _propagate_body,
        out_type=(
            jax.ShapeDtypeStruct((NP, D), f32),
            jax.ShapeDtypeStruct((N_STAGES * S, H), f32),
        ),
        mesh=mesh,
        compiler_params=pltpu.CompilerParams(use_tc_tiling_on_sc=False,
                                             needs_layout_passes=False),
        scratch_types=(
            pltpu.VMEM((B, CHUNK), jnp.int32),
            pltpu.VMEM((B, CHUNK), jnp.int32),
            pltpu.VMEM((B, CHUNK), f32),
            pltpu.VMEM((B, CHUNK), jnp.int32),
            pltpu.VMEM((B, CHUNK), jnp.int32),
            pltpu.VMEM((B, CHUNK), f32),
            pltpu.VMEM((B * CHUNK, H), f32),
            pltpu.VMEM((CHUNK, H), f32),
            pltpu.VMEM_SHARED((NP, H), f32),
            pltpu.SemaphoreType.DMA,
            pltpu.SemaphoreType.DMA,
            pltpu.SemaphoreType.DMA,
        ),
    )
    out, _tab = run(t0, dst, src2, val)
    return out


def kernel(user_emb, item_emb, edge_index, edge_values):
    zrows = jnp.zeros((NP - N, D), jnp.float32)
    t0 = jnp.concatenate([user_emb, item_emb, zrows], axis=0)
    dst = edge_index[0].astype(jnp.int32)
    src = edge_index[1].astype(jnp.int32)
    val = edge_values.astype(jnp.float32)
    pad = E_PAD - E
    dst = jnp.pad(dst, (0, pad))
    src = jnp.pad(src, (0, pad))
    val = jnp.pad(val, (0, pad))
    src2 = jnp.concatenate([src, src + NP])
    out = _propagate(t0, dst, src2, val)
    return (out[:N_USERS], out[N_USERS:N])


# speedup vs baseline: 18.2909x; 1.0837x over previous
import jax
import jax.numpy as jnp
from jax import lax
from jax.experimental import pallas as pl
from jax.experimental.pallas import tpu as pltpu
from jax.experimental.pallas import tpu_sc as plsc

N_USERS = 50000
N_ITEMS = 50000
N = N_USERS + N_ITEMS
NP = 100352
H = 16
D = 32
E = 1600000
NC = 2
NS = 16
CHUNK = 128
SG = 4
SGE = SG * CHUNK
B = 8
E_PER_TILE = -(-E // (NS * 4 * SGE)) * 4 * SGE
E_PAD = E_PER_TILE * NS
N_QUADS = E_PER_TILE // (4 * SGE)
ROWS_PER_TILE = NP // NS
S = NC * NP
N_STAGES = 4
INITC = 896
MEANC = 224


def _propagate_body(t0_r, dst_r, src_r, val_r,
                    out_r, tab_r,
                    dst0, dst1, dst2, dst3,
                    srcA, srcB, valA, valB,
                    rows,
                    zbuf,
                    acc,
                    sem_in, sem_g, sem_s):
    c = lax.axis_index("c")
    s = lax.axis_index("s")
    coff = c * NP
    row0 = s * ROWS_PER_TILE
    zero16 = jnp.zeros((16,), jnp.float32)
    ebase = s * E_PER_TILE
    cs = c * H

    def _z(i, _):
        zbuf[i] = zero16
        return 0
    lax.fori_loop(0, CHUNK, _z, 0)

    for k in range(ROWS_PER_TILE // INITC):
        r = row0 + k * INITC
        stage = rows.at[pl.ds(0, INITC)]
        pltpu.sync_copy(t0_r.at[pl.ds(r, INITC), pl.ds(cs, H)], stage)
        pltpu.sync_copy(stage, tab_r.at[pl.ds(coff + r, INITC)])

    def fire_drain(mk_src, mk_dst, n, sem):
        def _f(k, _):
            pltpu.async_copy(mk_src(k), mk_dst(k), sem)
            return 0
        lax.fori_loop(0, n, _f, 0)

        def _w(k, _):
            pltpu.make_async_copy(mk_src(k), mk_dst(k), sem).wait()
            return 0
        lax.fori_loop(0, n, _w, 0)

    n_zc = ROWS_PER_TILE // CHUNK

    def zero_acc():
        fire_drain(lambda k: zbuf,
                   lambda k: acc.at[pl.ds(row0 + k * CHUNK, CHUNK)],
                   n_zc, sem_s)

    def writeback(l):
        tb = (l + 1) * S + coff + row0
        fire_drain(lambda k: acc.at[pl.ds(row0 + k * CHUNK, CHUNK)],
                   lambda k: tab_r.at[pl.ds(tb + k * CHUNK, CHUNK)],
                   n_zc, sem_s)

    def layer(l, _):
        zero_acc()
        plsc.subcore_barrier()

        sbase = c * E_PAD + ebase
        tin = tab_r.at[pl.ds(l * S, S)]

        def issue_in(k, ds_, sb, vb):
            o = k * SGE
            for b in range(SG):
                off = o + b * CHUNK
                pltpu.async_copy(
                    dst_r.at[pl.ds(ebase + off, CHUNK)], ds_.at[b], sem_in)
                pltpu.async_copy(
                    src_r.at[pl.ds(sbase + off, CHUNK)], sb.at[b], sem_in)
                pltpu.async_copy(
                    val_r.at[pl.ds(ebase + off, CHUNK)], vb.at[b], sem_in)

        def drain_in(ds_, sb, vb):
            for b in range(SG):
                pltpu.make_async_copy(
                    dst_r.at[pl.ds(ebase, CHUNK)], ds_.at[b], sem_in).wait()
                pltpu.make_async_copy(
                    src_r.at[pl.ds(sbase, CHUNK)], sb.at[b], sem_in).wait()
                pltpu.make_async_copy(
                    val_r.at[pl.ds(ebase, CHUNK)], vb.at[b], sem_in).wait()

        def issue_g(sb, h):
            for b in range(SG):
                pltpu.async_copy(
                    tin.at[sb.at[b]],
                    rows.at[pl.ds((h * SG + b) * CHUNK, CHUNK)], sem_g)

        def drain_g(sb, h):
            for b in range(SG):
                pltpu.make_async_copy(
                    tin.at[sb.at[b]],
                    rows.at[pl.ds((h * SG + b) * CHUNK, CHUNK)],
                    sem_g).wait()

        def mult(vb, h):
            for b in range(SG):
                vslot = vb.at[b]
                for g in range(CHUNK // 16):
                    valg = vslot[pl.ds(g * 16, 16)]
                    for j in range(16):
                        e = (h * SG + b) * CHUNK + g * 16 + j
                        vbc = jnp.full((16,), valg[j])
                        rows[e] = rows[e] * vbc

        def issue_s(ds_, h):
            for b in range(SG):
                pltpu.async_copy(
                    rows.at[pl.ds((h * SG + b) * CHUNK, CHUNK)],
                    acc.at[ds_.at[b]], sem_s, add=True)

        def drain_s(ds_, h):
            for b in range(SG):
                pltpu.make_async_copy(
                    rows.at[pl.ds((h * SG + b) * CHUNK, CHUNK)],
                    acc.at[ds_.at[b]], sem_s).wait()

        dsts = (dst0, dst1, dst2, dst3)
        srcs = (srcA, srcB)
        vals = (valA, valB)

        issue_in(0, dst0, srcA, valA)
        issue_in(1, dst1, srcB, valB)
        drain_in(dst0, srcA, valA)
        issue_g(srcA, 0)

        def quad(qi, _):
            for u in range(4):
                k = 4 * qi + u
                h = u % 2
                h2 = 1 - h
                sb, vb = srcs[h], vals[h]
                sb2, vb2 = srcs[h2], vals[h2]
                dcur = dsts[u]
                dprev = dsts[(u - 1) % 4]
                dnext = dsts[(u + 1) % 4]
                dpre2 = dsts[(u + 2) % 4]
                if u == 0:
                    @pl.when(qi > 0)
                    def _():
                        drain_s(dprev, h2)
                else:
                    drain_s(dprev, h2)
                if u == 3:
                    @pl.when(qi < N_QUADS - 1)
                    def _():
                        drain_in(dnext, sb2, vb2)
                        issue_g(sb2, h2)
                else:
                    drain_in(dnext, sb2, vb2)
                    issue_g(sb2, h2)
                drain_g(sb, h)
                mult(vb, h)
                issue_s(dcur, h)
                if u >= 2:
                    @pl.when(qi < N_QUADS - 1)
                    def _():
                        issue_in(k + 2, dpre2, sb, vb)
                else:
                    issue_in(k + 2, dpre2, sb, vb)
            return 0

        lax.fori_loop(0, N_QUADS, quad, 0)
        drain_s(dst3, 1)
        plsc.subcore_barrier()

        writeback(l)
        plsc.subcore_barrier()
        return 0

    lax.fori_loop(0, N_LAYERS, layer, 0)

    def _mean(k, _):
        r = row0 + k * MEANC
        hb = coff + r
        d = []
        for l in range(N_STAGES):
            d.append(pltpu.async_copy(
                tab_r.at[pl.ds(l * S + hb, MEANC)],
                rows.at[pl.ds(l * MEANC, MEANC)], sem_in))
        for x in d:
            x.wait()

        def _m(j, _):
            acc_v = (rows[j] + rows[MEANC + j] +
                     rows[2 * MEANC + j] + rows[3 * MEANC + j]) * 0.25
            rows[j] = acc_v
            return 0
        lax.fori_loop(0, MEANC, _m, 0)
        pltpu.sync_copy(rows.at[pl.ds(0, MEANC)],
                        out_r.at[pl.ds(r, MEANC), pl.ds(cs, H)])
        return 0
    lax.fori_loop(0, ROWS_PER_TILE // MEANC, _mean, 0)


N_LAYERS = 3


@jax.jit
def _propagate(t0, dst, src2, val):
    mesh = plsc.VectorSubcoreMesh(core_axis_name="c", subcore_axis_name="s",
                                  num_cores=NC, num_subcores=NS)
    f32 = jnp.float32
    run = pl.kernel(
        _propagate_body,
        out_type=(
            jax.ShapeDtypeStruct((NP, D), f32),
            jax.ShapeDtypeStruct((N_STAGES * S, H), f32),
        ),
        mesh=mesh,
        compiler_params=pltpu.CompilerParams(use_tc_tiling_on_sc=False,
                                             needs_layout_passes=False),
        scratch_types=(
            pltpu.VMEM((SG, CHUNK), jnp.int32),
            pltpu.VMEM((SG, CHUNK), jnp.int32),
            pltpu.VMEM((SG, CHUNK), jnp.int32),
            pltpu.VMEM((SG, CHUNK), jnp.int32),
            pltpu.VMEM((SG, CHUNK), jnp.int32),
            pltpu.VMEM((SG, CHUNK), jnp.int32),
            pltpu.VMEM((SG, CHUNK), f32),
            pltpu.VMEM((SG, CHUNK), f32),
            pltpu.VMEM((B * CHUNK, H), f32),
            pltpu.VMEM((CHUNK, H), f32),
            pltpu.VMEM_SHARED((NP, H), f32),
            pltpu.SemaphoreType.DMA,
            pltpu.SemaphoreType.DMA,
            pltpu.SemaphoreType.DMA,
        ),
    )
    out, _tab = run(t0, dst, src2, val)
    return out


def kernel(user_emb, item_emb, edge_index, edge_values):
    zrows = jnp.zeros((NP - N, D), jnp.float32)
    t0 = jnp.concatenate([user_emb, item_emb, zrows], axis=0)
    dst = edge_index[0].astype(jnp.int32)
    src = edge_index[1].astype(jnp.int32)
    val = edge_values.astype(jnp.float32)
    pad = E_PAD - E
    dst = jnp.pad(dst, (0, pad))
    src = jnp.pad(src, (0, pad))
    val = jnp.pad(val, (0, pad))
    src2 = jnp.concatenate([src, src + NP])
    out = _propagate(t0, dst, src2, val)
    return (out[:N_USERS], out[N_USERS:N])


# speedup vs baseline: 19.6171x; 1.0725x over previous
import jax
import jax.numpy as jnp
from jax import lax
from jax.experimental import pallas as pl
from jax.experimental.pallas import tpu as pltpu
from jax.experimental.pallas import tpu_sc as plsc

N_USERS = 50000
N_ITEMS = 50000
N = N_USERS + N_ITEMS
NP = 100352
H = 16
D = 32
E = 1600000
NC = 2
NS = 16
CHUNK = 128
SG = 4
SGE = SG * CHUNK
B = 8
E_PER_TILE = -(-E // (NS * 4 * SGE)) * 4 * SGE
E_PAD = E_PER_TILE * NS
N_QUADS = E_PER_TILE // (4 * SGE)
ROWS_PER_TILE = NP // NS
S = NC * NP
N_STAGES = 4
INITC = 625
MEANC = 250


def _propagate_body(user_r, item_r, ei_r, val_r,
                    uout_r, iout_r, tab_r,
                    dst0, dst1, dst2, dst3,
                    srcA, srcB, valA, valB,
                    rows,
                    zbuf,
                    acc,
                    sem_in, sem_g, sem_s):
    c = lax.axis_index("c")
    s = lax.axis_index("s")
    coff = c * NP
    row0 = s * ROWS_PER_TILE
    zero16 = jnp.zeros((16,), jnp.float32)
    ebase = s * E_PER_TILE
    cs = c * H

    def _z(i, _):
        zbuf[i] = zero16
        return 0
    lax.fori_loop(0, CHUNK, _z, 0)

    NH = N // NS
    tr0 = (s % 8) * NH
    for half, src_tab in ((0, user_r), (1, item_r)):
        @pl.when((s // 8) == half)
        def _():
            for k in range(NH // INITC):
                r = tr0 + k * INITC
                stage = rows.at[pl.ds(0, INITC)]
                pltpu.sync_copy(src_tab.at[pl.ds(r, INITC), pl.ds(cs, H)],
                                stage)
                pltpu.sync_copy(
                    stage,
                    tab_r.at[pl.ds(coff + half * N_USERS + r, INITC)])

    def fire_drain(mk_src, mk_dst, n, sem):
        def _f(k, _):
            pltpu.async_copy(mk_src(k), mk_dst(k), sem)
            return 0
        lax.fori_loop(0, n, _f, 0)

        def _w(k, _):
            pltpu.make_async_copy(mk_src(k), mk_dst(k), sem).wait()
            return 0
        lax.fori_loop(0, n, _w, 0)

    n_zc = ROWS_PER_TILE // CHUNK

    def zero_acc():
        fire_drain(lambda k: zbuf,
                   lambda k: acc.at[pl.ds(row0 + k * CHUNK, CHUNK)],
                   n_zc, sem_s)

    def writeback(l):
        tb = (l + 1) * S + coff + row0
        fire_drain(lambda k: acc.at[pl.ds(row0 + k * CHUNK, CHUNK)],
                   lambda k: tab_r.at[pl.ds(tb + k * CHUNK, CHUNK)],
                   n_zc, sem_s)

    def layer(l, _):
        zero_acc()
        plsc.subcore_barrier()

        tin = tab_r.at[pl.ds(l * S, S)]

        def issue_in(k, ds_, sb, vb):
            o = k * SGE
            for b in range(SG):
                off = o + b * CHUNK
                pltpu.async_copy(
                    ei_r.at[pl.ds(ebase + off, CHUNK)], ds_.at[b], sem_in)
                pltpu.async_copy(
                    ei_r.at[pl.ds(E_PAD + ebase + off, CHUNK)], sb.at[b],
                    sem_in)
                pltpu.async_copy(
                    val_r.at[pl.ds(ebase + off, CHUNK)], vb.at[b], sem_in)

        def drain_in(ds_, sb, vb):
            for b in range(SG):
                pltpu.make_async_copy(
                    ei_r.at[pl.ds(ebase, CHUNK)], ds_.at[b], sem_in).wait()
                pltpu.make_async_copy(
                    ei_r.at[pl.ds(ebase, CHUNK)], sb.at[b], sem_in).wait()
                pltpu.make_async_copy(
                    val_r.at[pl.ds(ebase, CHUNK)], vb.at[b], sem_in).wait()
            for b in range(SG):
                sslot = sb.at[b]
                for g in range(CHUNK // 16):
                    sl = pl.ds(g * 16, 16)
                    sslot[sl] = sslot[sl] + coff

        def issue_g(sb, h):
            for b in range(SG):
                pltpu.async_copy(
                    tin.at[sb.at[b]],
                    rows.at[pl.ds((h * SG + b) * CHUNK, CHUNK)], sem_g)

        def drain_g(sb, h):
            for b in range(SG):
                pltpu.make_async_copy(
                    tin.at[sb.at[b]],
                    rows.at[pl.ds((h * SG + b) * CHUNK, CHUNK)],
                    sem_g).wait()

        def mult(vb, h):
            for b in range(SG):
                vslot = vb.at[b]
                for g in range(CHUNK // 16):
                    valg = vslot[pl.ds(g * 16, 16)]
                    for j in range(16):
                        e = (h * SG + b) * CHUNK + g * 16 + j
                        vbc = jnp.full((16,), valg[j])
                        rows[e] = rows[e] * vbc

        def issue_s(ds_, h):
            for b in range(SG):
                pltpu.async_copy(
                    rows.at[pl.ds((h * SG + b) * CHUNK, CHUNK)],
                    acc.at[ds_.at[b]], sem_s, add=True)

        def drain_s(ds_, h):
            for b in range(SG):
                pltpu.make_async_copy(
                    rows.at[pl.ds((h * SG + b) * CHUNK, CHUNK)],
                    acc.at[ds_.at[b]], sem_s).wait()

        dsts = (dst0, dst1, dst2, dst3)
        srcs = (srcA, srcB)
        vals = (valA, valB)

        issue_in(0, dst0, srcA, valA)
        issue_in(1, dst1, srcB, valB)
        drain_in(dst0, srcA, valA)
        issue_g(srcA, 0)

        def quad(qi, _):
            for u in range(4):
                k = 4 * qi + u
                h = u % 2
                h2 = 1 - h
                sb, vb = srcs[h], vals[h]
                sb2, vb2 = srcs[h2], vals[h2]
                dcur = dsts[u]
                dprev = dsts[(u - 1) % 4]
                dnext = dsts[(u + 1) % 4]
                dpre2 = dsts[(u + 2) % 4]
                if u == 0:
                    @pl.when(qi > 0)
                    def _():
                        drain_s(dprev, h2)
                else:
                    drain_s(dprev, h2)
                if u == 3:
                    @pl.when(qi < N_QUADS - 1)
                    def _():
                        drain_in(dnext, sb2, vb2)
                        issue_g(sb2, h2)
                else:
                    drain_in(dnext, sb2, vb2)
                    issue_g(sb2, h2)
                drain_g(sb, h)
                mult(vb, h)
                issue_s(dcur, h)
                if u >= 2:
                    @pl.when(qi < N_QUADS - 1)
                    def _():
                        issue_in(k + 2, dpre2, sb, vb)
                else:
                    issue_in(k + 2, dpre2, sb, vb)
            return 0

        lax.fori_loop(0, N_QUADS, quad, 0)
        drain_s(dst3, 1)
        plsc.subcore_barrier()

        writeback(l)
        plsc.subcore_barrier()
        return 0

    lax.fori_loop(0, N_LAYERS, layer, 0)

    for half, out_tab in ((0, uout_r), (1, iout_r)):
        @pl.when((s // 8) == half)
        def _():
            def _mean(k, _):
                r = tr0 + k * MEANC
                hb = coff + half * N_USERS + r
                d = []
                for l in range(N_STAGES):
                    d.append(pltpu.async_copy(
                        tab_r.at[pl.ds(l * S + hb, MEANC)],
                        rows.at[pl.ds(l * MEANC, MEANC)], sem_in))
                for x in d:
                    x.wait()

                def _m(j, _):
                    acc_v = (rows[j] + rows[MEANC + j] +
                             rows[2 * MEANC + j] + rows[3 * MEANC + j]) * 0.25
                    rows[j] = acc_v
                    return 0
                lax.fori_loop(0, MEANC, _m, 0)
                pltpu.sync_copy(rows.at[pl.ds(0, MEANC)],
                                out_tab.at[pl.ds(r, MEANC), pl.ds(cs, H)])
                return 0
            lax.fori_loop(0, NH // MEANC, _mean, 0)


N_LAYERS = 3


@jax.jit
def _propagate(user_emb, item_emb, ei, val):
    mesh = plsc.VectorSubcoreMesh(core_axis_name="c", subcore_axis_name="s",
                                  num_cores=NC, num_subcores=NS)
    f32 = jnp.float32
    run = pl.kernel(
        _propagate_body,
        out_type=(
            jax.ShapeDtypeStruct((N_USERS, D), f32),
            jax.ShapeDtypeStruct((N_ITEMS, D), f32),
            jax.ShapeDtypeStruct((N_STAGES * S, H), f32),
        ),
        mesh=mesh,
        compiler_params=pltpu.CompilerParams(use_tc_tiling_on_sc=False,
                                             needs_layout_passes=False),
        scratch_types=(
            pltpu.VMEM((SG, CHUNK), jnp.int32),
            pltpu.VMEM((SG, CHUNK), jnp.int32),
            pltpu.VMEM((SG, CHUNK), jnp.int32),
            pltpu.VMEM((SG, CHUNK), jnp.int32),
            pltpu.VMEM((SG, CHUNK), jnp.int32),
            pltpu.VMEM((SG, CHUNK), jnp.int32),
            pltpu.VMEM((SG, CHUNK), f32),
            pltpu.VMEM((SG, CHUNK), f32),
            pltpu.VMEM((B * CHUNK, H), f32),
            pltpu.VMEM((CHUNK, H), f32),
            pltpu.VMEM_SHARED((NP, H), f32),
            pltpu.SemaphoreType.DMA,
            pltpu.SemaphoreType.DMA,
            pltpu.SemaphoreType.DMA,
        ),
    )
    uo, io, _tab = run(user_emb, item_emb, ei, val)
    return uo, io


def kernel(user_emb, item_emb, edge_index, edge_values):
    ei = jnp.pad(edge_index.astype(jnp.int32),
                 ((0, 0), (0, E_PAD - E))).reshape(-1)
    val = jnp.pad(edge_values.astype(jnp.float32), (0, E_PAD - E))
    return _propagate(user_emb, item_emb, ei, val)
